# baseline (speedup 1.0000x reference)
import jax
import jax.numpy as jnp
from jax import lax
from jax.experimental import pallas as pl
from jax.experimental.pallas import tpu as pltpu
from jax.experimental.pallas import tpu_sc as plsc

N = 50000
D = 64
E = 800000
DH = D // 2

NS = 16
NP = 50176
SLAB = NP // NS
TRASH = 512
B = 128
NBJ = 8
ESZ = NBJ * B
ET = E // NS
NFULL = ET // ESZ
TAIL = ET - NFULL * ESZ
TAILC = TAIL // 16
RD = 4


def _sc_body(x2_h, adj_h, z2_h, z1_h, one_h, agg2_h, cnt_h,
             srcv, dstv1, dstv, gidx, rows, onev, acc, cnta,
             sem_e, sem_g, sem_s):
    c = lax.axis_index("c")
    s = lax.axis_index("s")
    slab = s * SLAB

    pltpu.sync_copy(z2_h, acc.at[pl.ds(slab, SLAB)])
    pltpu.sync_copy(z1_h, cnta.at[pl.ds(slab, SLAB)])
    pltpu.sync_copy(one_h, onev)
    plsc.subcore_barrier()

    ebase = s * ET

    def stage_compute(p):
        for j in range(NBJ):
            for k in range(B // 16):
                o = j * B + k * 16
                sv = srcv[p, pl.ds(o, 16)]
                gidx[j, pl.ds(k * 16, 16)] = sv + sv + c
                dstv[j, pl.ds(k * 16, 16)] = dstv1[p, pl.ds(o, 16)]

        gds = [None] * NBJ
        sds = [None] * NBJ
        for j in range(RD):
            gds[j] = pltpu.async_copy(x2_h.at[gidx.at[j]], rows.at[j], sem_g)
        for j in range(NBJ):
            b = j % RD
            gds[j].wait()
            sds[j] = pltpu.async_copy(rows.at[b], acc.at[dstv.at[j]],
                                      sem_s, add=True)

            @pl.when(c == 0)
            def _(j=j):
                pltpu.async_copy(onev, cnta.at[dstv.at[j]], sem_s, add=True)
            if j + RD < NBJ:
                sds[j].wait()
                gds[j + RD] = pltpu.async_copy(x2_h.at[gidx.at[j + RD]],
                                               rows.at[b], sem_g)
        for j in range(NBJ - RD, NBJ):
            sds[j].wait()

        @pl.when(c == 0)
        def _(p=p):
            for j in range(NBJ):
                pltpu.make_async_copy(onev, cnta.at[dstv.at[j]], sem_s).wait()

    pltpu.async_copy(adj_h.at[0, pl.ds(ebase, ESZ)], srcv.at[0], sem_e)
    pltpu.async_copy(adj_h.at[1, pl.ds(ebase, ESZ)], dstv1.at[0], sem_e)

    def pair_body(sp, carry):
        for p in range(2):
            st = sp * 2 + p
            pltpu.make_async_copy(adj_h.at[0, pl.ds(ebase, ESZ)],
                                  srcv.at[p], sem_e).wait()
            pltpu.make_async_copy(adj_h.at[1, pl.ds(ebase, ESZ)],
                                  dstv1.at[p], sem_e).wait()

            @pl.when(st + 1 < NFULL)
            def _():
                eoff = ebase + (st + 1) * ESZ
                pltpu.async_copy(adj_h.at[0, pl.ds(eoff, ESZ)],
                                 srcv.at[1 - p], sem_e)
                pltpu.async_copy(adj_h.at[1, pl.ds(eoff, ESZ)],
                                 dstv1.at[1 - p], sem_e)

            @pl.when(st + 1 == NFULL)
            def _():
                eoff = ebase + NFULL * ESZ
                pltpu.async_copy(adj_h.at[0, pl.ds(eoff, TAIL)],
                                 srcv.at[1 - p, pl.ds(0, TAIL)], sem_e)
                pltpu.async_copy(adj_h.at[1, pl.ds(eoff, TAIL)],
                                 dstv1.at[1 - p, pl.ds(0, TAIL)], sem_e)

            stage_compute(p)
        return carry

    lax.fori_loop(0, NFULL // 2, pair_body, 0)

    eoff = ebase + NFULL * ESZ
    pltpu.make_async_copy(adj_h.at[0, pl.ds(eoff, TAIL)],
                          srcv.at[0, pl.ds(0, TAIL)], sem_e).wait()
    pltpu.make_async_copy(adj_h.at[1, pl.ds(eoff, TAIL)],
                          dstv1.at[0, pl.ds(0, TAIL)], sem_e).wait()
    io = lax.iota(jnp.int32, 16)
    for k in range(TAILC, ESZ // 16):
        srcv[0, pl.ds(k * 16, 16)] = io * 631 + k * 13
        dstv1[0, pl.ds(k * 16, 16)] = NP + ((io + k * 16) & (TRASH - 1))
    stage_compute(0)

    plsc.subcore_barrier()

    pltpu.sync_copy(acc.at[pl.ds(slab, SLAB)], agg2_h.at[c, pl.ds(slab, SLAB)])

    @pl.when(c == 0)
    def _():
        pltpu.sync_copy(cnta.at[pl.ds(slab, SLAB)], cnt_h.at[pl.ds(slab, SLAB)])


_sc_aggregate = pl.kernel(
    _sc_body,
    out_type=(
        jax.ShapeDtypeStruct((2, NP, DH), jnp.float32),
        jax.ShapeDtypeStruct((NP,), jnp.float32),
    ),
    mesh=plsc.VectorSubcoreMesh(core_axis_name="c", subcore_axis_name="s"),
    scratch_types=[
        pltpu.VMEM((2, ESZ), jnp.int32),
        pltpu.VMEM((2, ESZ), jnp.int32),
        pltpu.VMEM((NBJ, B), jnp.int32),
        pltpu.VMEM((NBJ, B), jnp.int32),
        pltpu.VMEM((RD, B, DH), jnp.float32),
        pltpu.VMEM((B,), jnp.float32),
        pltpu.VMEM_SHARED((NP + TRASH, DH), jnp.float32),
        pltpu.VMEM_SHARED((NP + TRASH,), jnp.float32),
        pltpu.SemaphoreType.DMA,
        pltpu.SemaphoreType.DMA,
        pltpu.SemaphoreType.DMA,
    ],
    compiler_params=pltpu.CompilerParams(use_tc_tiling_on_sc=False),
)


def _tc_body(aggl_ref, aggr_ref, cnt_ref, x_ref, wl_ref, bl_ref, wr_ref,
             out_ref, z_ref):
    inv = 1.0 / jnp.maximum(cnt_ref[...], 1.0)
    wl = wl_ref[...]
    o = (jnp.dot(aggl_ref[0] * inv, wl[:DH, :],
                 preferred_element_type=jnp.float32)
         + jnp.dot(aggr_ref[0] * inv, wl[DH:, :],
                   preferred_element_type=jnp.float32)
         + bl_ref[...]
         + jnp.dot(x_ref[...], wr_ref[...],
                   preferred_element_type=jnp.float32))
    out_ref[...] = o
    m = jnp.max(o, axis=1, keepdims=True)
    z_ref[...] = o - (m + jnp.log(jnp.sum(jnp.exp(o - m), axis=1,
                                          keepdims=True)))


BR = 2000


def _tc_combine(agg2, cnt, x, w_l, b_l, w_r):
    grid = (N // BR,)
    return pl.pallas_call(
        _tc_body,
        grid=grid,
        in_specs=[
            pl.BlockSpec((1, BR, DH), lambda i: (0, i, 0)),
            pl.BlockSpec((1, BR, DH), lambda i: (1, i, 0)),
            pl.BlockSpec((BR, 1), lambda i: (i, 0)),
            pl.BlockSpec((BR, D), lambda i: (i, 0)),
            pl.BlockSpec((D, D), lambda i: (0, 0)),
            pl.BlockSpec((1, D), lambda i: (0, 0)),
            pl.BlockSpec((D, D), lambda i: (0, 0)),
        ],
        out_specs=[
            pl.BlockSpec((BR, D), lambda i: (i, 0)),
            pl.BlockSpec((BR, D), lambda i: (i, 0)),
        ],
        out_shape=[
            jax.ShapeDtypeStruct((N, D), jnp.float32),
            jax.ShapeDtypeStruct((N, D), jnp.float32),
        ],
    )(agg2, agg2, cnt, x, w_l, b_l, w_r)


@jax.jit
def kernel(x, adj_t, W_l, b_l, W_r):
    adj = adj_t.astype(jnp.int32)
    x2 = x.reshape(2 * N, DH)
    z2 = jnp.zeros((SLAB, DH), jnp.float32)
    z1 = jnp.zeros((SLAB,), jnp.float32)
    ones = jnp.ones((B,), jnp.float32)
    agg2, cnt = _sc_aggregate(x2, adj, z2, z1, ones)
    out, z = _tc_combine(agg2, cnt.reshape(NP, 1), x, W_l,
                         b_l.reshape(1, D), W_r)
    return (out, z)


# speedup vs baseline: 1.0056x; 1.0056x over previous
import jax
import jax.numpy as jnp
from jax import lax
from jax.experimental import pallas as pl
from jax.experimental.pallas import tpu as pltpu
from jax.experimental.pallas import tpu_sc as plsc

N = 50000
D = 64
E = 800000
DH = D // 2

NS = 16
NP = 50176
SLAB = NP // NS
TRASH = 512
B = 128
NBJ = 8
ESZ = NBJ * B
ET = E // NS
NFULL = ET // ESZ
TAIL = ET - NFULL * ESZ
TAILC = TAIL // 16
RD = 4


def _sc_body(x2_h, adj_h, z2_h, z1_h, one_h, agg2_h, cnt_h,
             srcv, dstv1, dstv, gidx, rows, onev, acc, cnta,
             sem_e, sem_g, sem_s):
    c = lax.axis_index("c")
    s = lax.axis_index("s")
    slab = s * SLAB

    pltpu.sync_copy(z2_h, acc.at[pl.ds(slab, SLAB)])
    pltpu.sync_copy(z1_h, cnta.at[pl.ds(slab, SLAB)])
    pltpu.sync_copy(one_h, onev)
    plsc.subcore_barrier()

    ebase = s * ET

    def stage_compute(p):
        for j in range(NBJ):
            for k in range(B // 16):
                o = j * B + k * 16
                sv = srcv[p, pl.ds(o, 16)]
                gidx[j, pl.ds(k * 16, 16)] = sv + sv + c
                dstv[j, pl.ds(k * 16, 16)] = dstv1[p, pl.ds(o, 16)]

        gds = [None] * NBJ
        sds = [None] * NBJ
        for j in range(RD):
            gds[j] = pltpu.async_copy(x2_h.at[gidx.at[j]], rows.at[j], sem_g)
        for j in range(NBJ):
            b = j % RD
            gds[j].wait()
            sds[j] = pltpu.async_copy(rows.at[b], acc.at[dstv.at[j]],
                                      sem_s, add=True)

            @pl.when(c == 0)
            def _(j=j):
                pltpu.async_copy(onev, cnta.at[dstv.at[j]], sem_s, add=True)
            if j + RD < NBJ:
                sds[j].wait()
                gds[j + RD] = pltpu.async_copy(x2_h.at[gidx.at[j + RD]],
                                               rows.at[b], sem_g)
        for j in range(NBJ - RD, NBJ):
            sds[j].wait()

        @pl.when(c == 0)
        def _(p=p):
            for j in range(NBJ):
                pltpu.make_async_copy(onev, cnta.at[dstv.at[j]], sem_s).wait()

    pltpu.async_copy(adj_h.at[0, pl.ds(ebase, ESZ)], srcv.at[0], sem_e)
    pltpu.async_copy(adj_h.at[1, pl.ds(ebase, ESZ)], dstv1.at[0], sem_e)

    def pair_body(sp, carry):
        for p in range(2):
            st = sp * 2 + p
            pltpu.make_async_copy(adj_h.at[0, pl.ds(ebase, ESZ)],
                                  srcv.at[p], sem_e).wait()
            pltpu.make_async_copy(adj_h.at[1, pl.ds(ebase, ESZ)],
                                  dstv1.at[p], sem_e).wait()

            @pl.when(st + 1 < NFULL)
            def _():
                eoff = ebase + (st + 1) * ESZ
                pltpu.async_copy(adj_h.at[0, pl.ds(eoff, ESZ)],
                                 srcv.at[1 - p], sem_e)
                pltpu.async_copy(adj_h.at[1, pl.ds(eoff, ESZ)],
                                 dstv1.at[1 - p], sem_e)

            @pl.when(st + 1 == NFULL)
            def _():
                eoff = ebase + NFULL * ESZ
                pltpu.async_copy(adj_h.at[0, pl.ds(eoff, TAIL)],
                                 srcv.at[1 - p, pl.ds(0, TAIL)], sem_e)
                pltpu.async_copy(adj_h.at[1, pl.ds(eoff, TAIL)],
                                 dstv1.at[1 - p, pl.ds(0, TAIL)], sem_e)

            stage_compute(p)
        return carry

    lax.fori_loop(0, NFULL // 2, pair_body, 0)

    eoff = ebase + NFULL * ESZ
    pltpu.make_async_copy(adj_h.at[0, pl.ds(eoff, TAIL)],
                          srcv.at[0, pl.ds(0, TAIL)], sem_e).wait()
    pltpu.make_async_copy(adj_h.at[1, pl.ds(eoff, TAIL)],
                          dstv1.at[0, pl.ds(0, TAIL)], sem_e).wait()
    io = lax.iota(jnp.int32, 16)
    for k in range(TAILC, ESZ // 16):
        srcv[0, pl.ds(k * 16, 16)] = io * 631 + k * 13
        dstv1[0, pl.ds(k * 16, 16)] = NP + ((io + k * 16) & (TRASH - 1))
    stage_compute(0)

    plsc.subcore_barrier()

    pltpu.sync_copy(acc.at[pl.ds(slab, SLAB)], agg2_h.at[c, pl.ds(slab, SLAB)])

    @pl.when(c == 0)
    def _():
        pltpu.sync_copy(cnta.at[pl.ds(slab, SLAB)], cnt_h.at[pl.ds(slab, SLAB)])


_sc_aggregate = pl.kernel(
    _sc_body,
    out_type=(
        jax.ShapeDtypeStruct((2, NP, DH), jnp.float32),
        jax.ShapeDtypeStruct((NP,), jnp.float32),
    ),
    mesh=plsc.VectorSubcoreMesh(core_axis_name="c", subcore_axis_name="s"),
    scratch_types=[
        pltpu.VMEM((2, ESZ), jnp.int32),
        pltpu.VMEM((2, ESZ), jnp.int32),
        pltpu.VMEM((NBJ, B), jnp.int32),
        pltpu.VMEM((NBJ, B), jnp.int32),
        pltpu.VMEM((RD, B, DH), jnp.float32),
        pltpu.VMEM((B,), jnp.float32),
        pltpu.VMEM_SHARED((NP + TRASH, DH), jnp.float32),
        pltpu.VMEM_SHARED((NP + TRASH,), jnp.float32),
        pltpu.SemaphoreType.DMA,
        pltpu.SemaphoreType.DMA,
        pltpu.SemaphoreType.DMA,
    ],
    compiler_params=pltpu.CompilerParams(use_tc_tiling_on_sc=False),
)


def _tc_body(aggl_ref, aggr_ref, cnt_ref, x_ref, wl_ref, bl_ref, wr_ref,
             out_ref, z_ref):
    inv = 1.0 / jnp.maximum(cnt_ref[...], 1.0)
    wl = wl_ref[...]
    o = (jnp.dot(aggl_ref[0] * inv, wl[:DH, :],
                 preferred_element_type=jnp.float32)
         + jnp.dot(aggr_ref[0] * inv, wl[DH:, :],
                   preferred_element_type=jnp.float32)
         + bl_ref[...]
         + jnp.dot(x_ref[...], wr_ref[...],
                   preferred_element_type=jnp.float32))
    out_ref[...] = o
    m = jnp.max(o, axis=1, keepdims=True)
    z_ref[...] = o - (m + jnp.log(jnp.sum(jnp.exp(o - m), axis=1,
                                          keepdims=True)))


BR = 5000


def _tc_combine(agg2, cnt, x, w_l, b_l, w_r):
    grid = (N // BR,)
    return pl.pallas_call(
        _tc_body,
        grid=grid,
        in_specs=[
            pl.BlockSpec((1, BR, DH), lambda i: (0, i, 0)),
            pl.BlockSpec((1, BR, DH), lambda i: (1, i, 0)),
            pl.BlockSpec((BR, 1), lambda i: (i, 0)),
            pl.BlockSpec((BR, D), lambda i: (i, 0)),
            pl.BlockSpec((D, D), lambda i: (0, 0)),
            pl.BlockSpec((1, D), lambda i: (0, 0)),
            pl.BlockSpec((D, D), lambda i: (0, 0)),
        ],
        out_specs=[
            pl.BlockSpec((BR, D), lambda i: (i, 0)),
            pl.BlockSpec((BR, D), lambda i: (i, 0)),
        ],
        out_shape=[
            jax.ShapeDtypeStruct((N, D), jnp.float32),
            jax.ShapeDtypeStruct((N, D), jnp.float32),
        ],
    )(agg2, agg2, cnt, x, w_l, b_l, w_r)


@jax.jit
def kernel(x, adj_t, W_l, b_l, W_r):
    adj = adj_t.astype(jnp.int32)
    x2 = x.reshape(2 * N, DH)
    z2 = jnp.zeros((SLAB, DH), jnp.float32)
    z1 = jnp.zeros((SLAB,), jnp.float32)
    ones = jnp.ones((B,), jnp.float32)
    agg2, cnt = _sc_aggregate(x2, adj, z2, z1, ones)
    out, z = _tc_combine(agg2, cnt.reshape(NP, 1), x, W_l,
                         b_l.reshape(1, D), W_r)
    return (out, z)
